# baseline (speedup 1.0000x reference)
import functools

import jax
import jax.numpy as jnp
from jax import lax
from jax.experimental import pallas as pl
from jax.experimental.pallas import tpu as pltpu
from jax.experimental.pallas import tpu_sc as plsc

B = 16384
N_FIELDS = 26
VOCAB = 100000
EMBED_DIM = 64
N_DENSE = 13
H1 = 1024
H2 = 512
EIN = N_FIELDS * EMBED_DIM

NW = 32
CHUNK = 128
RPW = B // NW
CPF = RPW // CHUNK
NCH = N_FIELDS * CPF
IPW = N_FIELDS * RPW


def _gather_body(xvt_hbm, table_hbm, out_hbm, idx_fm, rows_v, gsem, wsem):
    wid = lax.axis_index("c") * 16 + lax.axis_index("s")
    b0w = wid * RPW

    pltpu.sync_copy(xvt_hbm.at[:, pl.ds(b0w, RPW)], idx_fm)

    def to_super(j, _):
        for f in range(N_FIELDS):
            v = idx_fm[f, pl.ds(j * 16, 16)]
            idx_fm[f, pl.ds(j * 16, 16)] = lax.shift_right_logical(v, 1)
        return 0

    lax.fori_loop(0, RPW // 16, to_super, 0)

    def chunk_src(c):
        f = c // CPF
        b0 = (c % CPF) * CHUNK
        return f, b0

    def start_gather(c, slot):
        f, b0 = chunk_src(c)
        pltpu.async_copy(
            table_hbm.at[f].at[idx_fm.at[f, pl.ds(b0, CHUNK)]],
            rows_v.at[slot],
            gsem.at[slot])

    def start_writeback(c, slot):
        f, b0 = chunk_src(c)
        pltpu.async_copy(
            rows_v.at[slot],
            out_hbm.at[f, pl.ds(b0w + b0, CHUNK), :],
            wsem.at[slot])

    def wait(sem, slot):
        pltpu.make_async_copy(
            out_hbm.at[0, pl.ds(0, CHUNK), :], rows_v.at[slot],
            sem.at[slot]
        ).wait()

    start_gather(0, 0)

    def step(c, _):
        slot = lax.rem(c, 2)
        nslot = 1 - slot
        @pl.when(c + 1 < NCH)
        def _():
            @pl.when(c + 1 >= 2)
            def _():
                wait(wsem, nslot)
            start_gather(c + 1, nslot)
        wait(gsem, slot)
        start_writeback(c, slot)
        return 0

    lax.fori_loop(0, NCH, step, 0)
    wait(wsem, 0)
    wait(wsem, 1)


_gather = functools.partial(
    pl.kernel,
    mesh=plsc.VectorSubcoreMesh(core_axis_name="c", subcore_axis_name="s"),
    out_type=jax.ShapeDtypeStruct((N_FIELDS, B, 2 * EMBED_DIM), jnp.float32),
    scratch_types=[
        pltpu.VMEM((N_FIELDS, RPW), jnp.int32),
        pltpu.VMEM((2, CHUNK, 2 * EMBED_DIM), jnp.float32),
        pltpu.SemaphoreType.DMA((2,)),
        pltpu.SemaphoreType.DMA((2,)),
    ],
)(_gather_body)


BB = 512


def _mlp_body(xi_ref, emb_ref, par_ref, w1d_ref, w1e_ref, b1_ref, w2_ref,
              b2_ref, w3_ref, b3_ref, o_ref):
    fields = []
    for f in range(N_FIELDS):
        pair = emb_ref[f]
        sel = jnp.where(par_ref[:, f:f + 1] > 0.5,
                        pair[:, EMBED_DIM:], pair[:, :EMBED_DIM])
        fields.append(sel)
    x = jnp.concatenate(fields, axis=-1)
    h1 = jnp.dot(x, w1e_ref[...], preferred_element_type=jnp.float32)
    h1 = h1 + jnp.dot(xi_ref[...], w1d_ref[...], preferred_element_type=jnp.float32)
    h1 = jnp.maximum(h1 + b1_ref[...], 0.0)
    h2 = jnp.dot(h1, w2_ref[...], preferred_element_type=jnp.float32)
    h2 = jnp.maximum(h2 + b2_ref[...], 0.0)
    y = jnp.dot(h2, w3_ref[...], preferred_element_type=jnp.float32) + b3_ref[...]
    o_ref[...] = jax.nn.sigmoid(y)


def kernel(xi, xv, emb_tables, W1, b1, W2, b2, W3, b3):
    xvi = xv.astype(jnp.int32)
    xvt = xvi.T
    table3 = emb_tables.reshape(N_FIELDS, VOCAB // 2, 2 * EMBED_DIM)

    emb_pairs = _gather(xvt, table3)
    parity = (xvi & 1).astype(jnp.float32)

    W1d = W1[:N_DENSE]
    W1e = W1[N_DENSE:]

    out = pl.pallas_call(
        _mlp_body,
        grid=(B // BB,),
        in_specs=[
            pl.BlockSpec((BB, N_DENSE), lambda i: (i, 0)),
            pl.BlockSpec((N_FIELDS, BB, 2 * EMBED_DIM), lambda i: (0, i, 0)),
            pl.BlockSpec((BB, N_FIELDS), lambda i: (i, 0)),
            pl.BlockSpec((N_DENSE, H1), lambda i: (0, 0)),
            pl.BlockSpec((EIN, H1), lambda i: (0, 0)),
            pl.BlockSpec((1, H1), lambda i: (0, 0)),
            pl.BlockSpec((H1, H2), lambda i: (0, 0)),
            pl.BlockSpec((1, H2), lambda i: (0, 0)),
            pl.BlockSpec((H2, 1), lambda i: (0, 0)),
            pl.BlockSpec((1, 1), lambda i: (0, 0)),
        ],
        out_specs=pl.BlockSpec((BB, 1), lambda i: (i, 0)),
        out_shape=jax.ShapeDtypeStruct((B, 1), jnp.float32),
    )(xi, emb_pairs, parity, W1d, W1e, b1.reshape(1, H1), W2,
      b2.reshape(1, H2), W3, b3.reshape(1, 1))
    return out


# speedup vs baseline: 1.0036x; 1.0036x over previous
import functools

import jax
import jax.numpy as jnp
from jax import lax
from jax.experimental import pallas as pl
from jax.experimental.pallas import tpu as pltpu
from jax.experimental.pallas import tpu_sc as plsc

B = 16384
N_FIELDS = 26
VOCAB = 100000
EMBED_DIM = 64
N_DENSE = 13
H1 = 1024
H2 = 512
EIN = N_FIELDS * EMBED_DIM

NW = 32
CHUNK = 128
RPW = B // NW
CPF = RPW // CHUNK
NCH = N_FIELDS * CPF
IPW = N_FIELDS * RPW


def _gather_body(xvt_hbm, table_hbm, out_hbm, idx_fm, rows_v, gsem, wsem):
    wid = lax.axis_index("c") * 16 + lax.axis_index("s")
    b0w = wid * RPW

    pltpu.sync_copy(xvt_hbm.at[:, pl.ds(b0w, RPW)], idx_fm)

    def to_super(j, _):
        for f in range(N_FIELDS):
            v = idx_fm[f, pl.ds(j * 16, 16)]
            idx_fm[f, pl.ds(j * 16, 16)] = lax.shift_right_logical(v, 1)
        return 0

    lax.fori_loop(0, RPW // 16, to_super, 0)

    def chunk_src(c):
        f = c // CPF
        b0 = (c % CPF) * CHUNK
        return f, b0

    def start_gather(c, slot):
        f, b0 = chunk_src(c)
        pltpu.async_copy(
            table_hbm.at[f].at[idx_fm.at[f, pl.ds(b0, CHUNK)]],
            rows_v.at[slot],
            gsem.at[slot])

    def start_writeback(c, slot):
        f, b0 = chunk_src(c)
        pltpu.async_copy(
            rows_v.at[slot],
            out_hbm.at[f, pl.ds(b0w + b0, CHUNK), :],
            wsem.at[slot])

    def wait(sem, slot):
        pltpu.make_async_copy(
            out_hbm.at[0, pl.ds(0, CHUNK), :], rows_v.at[slot],
            sem.at[slot]
        ).wait()

    start_gather(0, 0)

    def step(c, _):
        slot = lax.rem(c, 2)
        nslot = 1 - slot
        @pl.when(c + 1 < NCH)
        def _():
            @pl.when(c + 1 >= 2)
            def _():
                wait(wsem, nslot)
            start_gather(c + 1, nslot)
        wait(gsem, slot)
        start_writeback(c, slot)
        return 0

    lax.fori_loop(0, NCH, step, 0)
    wait(wsem, 0)
    wait(wsem, 1)


_gather = functools.partial(
    pl.kernel,
    mesh=plsc.VectorSubcoreMesh(core_axis_name="c", subcore_axis_name="s"),
    out_type=jax.ShapeDtypeStruct((N_FIELDS, B, 2 * EMBED_DIM), jnp.float32),
    scratch_types=[
        pltpu.VMEM((N_FIELDS, RPW), jnp.int32),
        pltpu.VMEM((2, CHUNK, 2 * EMBED_DIM), jnp.float32),
        pltpu.SemaphoreType.DMA((2,)),
        pltpu.SemaphoreType.DMA((2,)),
    ],
)(_gather_body)


BB = 512


def _mlp_body(xi_ref, emb_ref, par_ref, w1d_ref, w1e_ref, b1_ref, w2_ref,
              b2_ref, w3_ref, b3_ref, o_ref):
    fields = []
    for f in range(N_FIELDS):
        pair = emb_ref[f]
        sel = jnp.where(par_ref[:, f:f + 1] > 0.5,
                        pair[:, EMBED_DIM:], pair[:, :EMBED_DIM])
        fields.append(sel.astype(jnp.bfloat16))
    x = jnp.concatenate(fields, axis=-1)
    h1 = jnp.dot(x, w1e_ref[...], preferred_element_type=jnp.float32)
    h1 = h1 + jnp.dot(xi_ref[...], w1d_ref[...], preferred_element_type=jnp.float32)
    h1 = jnp.maximum(h1 + b1_ref[...], 0.0).astype(jnp.bfloat16)
    h2 = jnp.dot(h1, w2_ref[...], preferred_element_type=jnp.float32)
    h2 = jnp.maximum(h2 + b2_ref[...], 0.0).astype(jnp.bfloat16)
    y = jnp.dot(h2, w3_ref[...], preferred_element_type=jnp.float32) + b3_ref[...]
    o_ref[...] = jax.nn.sigmoid(y)


def kernel(xi, xv, emb_tables, W1, b1, W2, b2, W3, b3):
    xvi = xv.astype(jnp.int32)
    xvt = xvi.T
    table3 = emb_tables.reshape(N_FIELDS, VOCAB // 2, 2 * EMBED_DIM)

    emb_pairs = _gather(xvt, table3)
    parity = (xvi & 1).astype(jnp.float32)

    W1d = W1[:N_DENSE].astype(jnp.bfloat16)
    W1e = W1[N_DENSE:].astype(jnp.bfloat16)
    W2h = W2.astype(jnp.bfloat16)
    W3h = W3.astype(jnp.bfloat16)
    xih = xi.astype(jnp.bfloat16)

    out = pl.pallas_call(
        _mlp_body,
        grid=(B // BB,),
        in_specs=[
            pl.BlockSpec((BB, N_DENSE), lambda i: (i, 0)),
            pl.BlockSpec((N_FIELDS, BB, 2 * EMBED_DIM), lambda i: (0, i, 0)),
            pl.BlockSpec((BB, N_FIELDS), lambda i: (i, 0)),
            pl.BlockSpec((N_DENSE, H1), lambda i: (0, 0)),
            pl.BlockSpec((EIN, H1), lambda i: (0, 0)),
            pl.BlockSpec((1, H1), lambda i: (0, 0)),
            pl.BlockSpec((H1, H2), lambda i: (0, 0)),
            pl.BlockSpec((1, H2), lambda i: (0, 0)),
            pl.BlockSpec((H2, 1), lambda i: (0, 0)),
            pl.BlockSpec((1, 1), lambda i: (0, 0)),
        ],
        out_specs=pl.BlockSpec((BB, 1), lambda i: (i, 0)),
        out_shape=jax.ShapeDtypeStruct((B, 1), jnp.float32),
    )(xih, emb_pairs, parity, W1d, W1e, b1.reshape(1, H1), W2h,
      b2.reshape(1, H2), W3h, b3.reshape(1, 1))
    return out
